# baseline (speedup 1.0000x reference)
import jax
import jax.numpy as jnp
from jax.experimental import pallas as pl
from jax.experimental.pallas import tpu as pltpu


def _block_sizes(seq):
    sizes = [256]
    while sum(sizes) < seq:
        sizes.append(min(sizes[-1] * 2 if len(sizes) > 1 else 256, seq - sum(sizes)))
    return tuple(sizes)


def _bcast_pipeline(sizes, tab_ref, out_ref, buf, in_sem, out_sem):
    batch = out_ref.shape[0]
    offs = [0]
    for s in sizes[:-1]:
        offs.append(offs[-1] + s)
    n = len(sizes)

    def in_copy(i):
        o, s = offs[i], sizes[i]
        return pltpu.make_async_copy(
            tab_ref.at[pl.ds(o, s), :], buf.at[pl.ds(o, s), :], in_sem.at[i]
        )

    def out_copy(i, b):
        o, s = offs[i], sizes[i]
        return pltpu.make_async_copy(
            buf.at[pl.ds(o, s), :], out_ref.at[b, pl.ds(o, s), :], out_sem.at[i, b]
        )

    in_copy(0).start()
    if n > 1:
        in_copy(1).start()
    for k in range(n):
        if k + 2 < n:
            in_copy(k + 2).start()
        in_copy(k).wait()
        for b in range(batch):
            out_copy(k, b).start()
    for i in range(n):
        for b in range(batch):
            out_copy(i, b).wait()


def kernel(input_ids, table):
    batch, seq = input_ids.shape
    hidden = table.shape[1]
    sizes = _block_sizes(seq)
    out = pl.pallas_call(
        lambda *refs: _bcast_pipeline(sizes, *refs),
        in_specs=[pl.BlockSpec(memory_space=pl.ANY)],
        out_specs=pl.BlockSpec(memory_space=pl.ANY),
        out_shape=jax.ShapeDtypeStruct((batch, seq, hidden), table.dtype),
        scratch_shapes=[
            pltpu.VMEM((seq, hidden), table.dtype),
            pltpu.SemaphoreType.DMA((len(sizes),)),
            pltpu.SemaphoreType.DMA((len(sizes), 4)),
        ],
    )(table)
    return out


# speedup vs baseline: 1.0110x; 1.0110x over previous
import jax
import jax.numpy as jnp
from jax.experimental import pallas as pl
from jax.experimental.pallas import tpu as pltpu


def _block_sizes(seq):
    if seq == 8192:
        return (1024, 3072, 4096)
    sizes = [256]
    while sum(sizes) < seq:
        sizes.append(min(sizes[-1] * 2 if len(sizes) > 1 else 256, seq - sum(sizes)))
    return tuple(sizes)


def _bcast_pipeline(sizes, tab_ref, out_ref, buf, in_sem, out_sem):
    batch = out_ref.shape[0]
    offs = [0]
    for s in sizes[:-1]:
        offs.append(offs[-1] + s)
    n = len(sizes)

    def in_copy(i):
        o, s = offs[i], sizes[i]
        return pltpu.make_async_copy(
            tab_ref.at[pl.ds(o, s), :], buf.at[pl.ds(o, s), :], in_sem.at[i]
        )

    def out_copy(i, b):
        o, s = offs[i], sizes[i]
        return pltpu.make_async_copy(
            buf.at[pl.ds(o, s), :], out_ref.at[b, pl.ds(o, s), :], out_sem.at[i, b]
        )

    in_copy(0).start()
    if n > 1:
        in_copy(1).start()
    for k in range(n):
        if k + 2 < n:
            in_copy(k + 2).start()
        in_copy(k).wait()
        for b in range(batch):
            out_copy(k, b).start()
    for i in range(n):
        for b in range(batch):
            out_copy(i, b).wait()


def kernel(input_ids, table):
    batch, seq = input_ids.shape
    hidden = table.shape[1]
    sizes = _block_sizes(seq)
    out = pl.pallas_call(
        lambda *refs: _bcast_pipeline(sizes, *refs),
        in_specs=[pl.BlockSpec(memory_space=pl.ANY)],
        out_specs=pl.BlockSpec(memory_space=pl.ANY),
        out_shape=jax.ShapeDtypeStruct((batch, seq, hidden), table.dtype),
        scratch_shapes=[
            pltpu.VMEM((seq, hidden), table.dtype),
            pltpu.SemaphoreType.DMA((len(sizes),)),
            pltpu.SemaphoreType.DMA((len(sizes), 4)),
        ],
    )(table)
    return out


# speedup vs baseline: 1.0171x; 1.0061x over previous
import jax
import jax.numpy as jnp
from jax.experimental import pallas as pl
from jax.experimental.pallas import tpu as pltpu


def _block_sizes(seq):
    if seq == 8192:
        return (2048, 6144)
    sizes = [256]
    while sum(sizes) < seq:
        sizes.append(min(sizes[-1] * 2 if len(sizes) > 1 else 256, seq - sum(sizes)))
    return tuple(sizes)


def _bcast_pipeline(sizes, tab_ref, out_ref, buf, in_sem, out_sem):
    batch = out_ref.shape[0]
    offs = [0]
    for s in sizes[:-1]:
        offs.append(offs[-1] + s)
    n = len(sizes)

    def in_copy(i):
        o, s = offs[i], sizes[i]
        return pltpu.make_async_copy(
            tab_ref.at[pl.ds(o, s), :], buf.at[pl.ds(o, s), :], in_sem.at[i]
        )

    def out_copy(i, b):
        o, s = offs[i], sizes[i]
        return pltpu.make_async_copy(
            buf.at[pl.ds(o, s), :], out_ref.at[b, pl.ds(o, s), :], out_sem.at[i, b]
        )

    in_copy(0).start()
    if n > 1:
        in_copy(1).start()
    for k in range(n):
        if k + 2 < n:
            in_copy(k + 2).start()
        in_copy(k).wait()
        for b in range(batch):
            out_copy(k, b).start()
    for i in range(n):
        for b in range(batch):
            out_copy(i, b).wait()


def kernel(input_ids, table):
    batch, seq = input_ids.shape
    hidden = table.shape[1]
    sizes = _block_sizes(seq)
    out = pl.pallas_call(
        lambda *refs: _bcast_pipeline(sizes, *refs),
        in_specs=[pl.BlockSpec(memory_space=pl.ANY)],
        out_specs=pl.BlockSpec(memory_space=pl.ANY),
        out_shape=jax.ShapeDtypeStruct((batch, seq, hidden), table.dtype),
        scratch_shapes=[
            pltpu.VMEM((seq, hidden), table.dtype),
            pltpu.SemaphoreType.DMA((len(sizes),)),
            pltpu.SemaphoreType.DMA((len(sizes), 4)),
        ],
    )(table)
    return out
